# baseline (speedup 1.0000x reference)
import functools
import math

import jax
import jax.numpy as jnp
import numpy as np
from jax import lax
from jax.experimental import pallas as pl
from jax.experimental.pallas import tpu as pltpu
from jax.experimental.pallas import tpu_sc as plsc

N = 10000
E = 320000
D = 128
K = 5000

NW = 32
NLOC = 320
NPAD = NW * NLOC
LCAP = 12800
WIN = 2000
NVEC = WIN // 16
NWIN = E // WIN
CH = 128
BIG = np.int32(2**30)

_HALF = [10080 * k for k in range(1, 12)] + [120720, 130560, 140400, 150240]
_CUTS = _HALF + [160000] + [160000 + c for c in _HALF]


def _scan_body(dst_hbm, src_hbm, lists_rel, lists_src, counts, dstb, srcb,
               lrel, lsrc):
    wid = lax.axis_index("s") * 2 + lax.axis_index("c")
    base = wid * NLOC
    nloc = jnp.minimum(np.int32(NLOC), np.int32(N) - base)
    lane = lax.iota(jnp.int32, 16)

    trash_v = jnp.zeros((16,), jnp.int32) + np.int32(NLOC + 15)

    def zero_body(j, _):
        lrel[pl.ds(j * 16, 16)] = trash_v
        lsrc[pl.ds(j * 16, 16)] = jnp.zeros((16,), jnp.int32)
        return 0

    lax.fori_loop(0, (LCAP + 16) // 16, zero_body, 0)

    def win_body(w, cur):
        pltpu.sync_copy(dst_hbm.at[pl.ds(w * WIN, WIN)], dstb)
        pltpu.sync_copy(src_hbm.at[pl.ds(w * WIN, WIN)], srcb)

        def vec_body(j, cur):
            dv = dstb[pl.ds(j * 16, 16)]
            rel = dv - base
            m = (rel >= 0) & (rel < nloc)
            pc = plsc.all_reduce_population_count(m)[0]

            @pl.when(pc > 0)
            def _():
                sv = srcb[pl.ds(j * 16, 16)]
                mi = jnp.where(m, 1, 0).astype(jnp.int32)
                cs = plsc.cumsum(mi)
                pos = jnp.where(m, cur + cs - 1, LCAP + lane)
                plsc.store_scatter(lrel, [pos], rel)
                plsc.store_scatter(lsrc, [pos], sv)

            return cur + pc

        return lax.fori_loop(0, NVEC, vec_body, cur)

    m_t = lax.fori_loop(0, NWIN, win_body, np.int32(0))

    pltpu.sync_copy(lrel.at[pl.ds(0, LCAP)], lists_rel.at[wid])
    pltpu.sync_copy(lsrc.at[pl.ds(0, LCAP)], lists_src.at[wid])
    for j in range(8):
        dstb[pl.ds(j * 16, 16)] = jnp.zeros((16,), jnp.int32) + m_t
    pltpu.sync_copy(dstb.at[pl.ds(0, 128)], counts.at[wid])


def _acc_body(lists_rel, lists_src, counts, x_hbm, agg_hbm, lrel, lsrc, gbuf,
              pacc, cntb, hist_sm, cnt_sm, brk_sm, bds_sm, cuts_sm, sem):
    wid = lax.axis_index("s") * 2 + lax.axis_index("c")
    base = wid * NLOC

    b_t = np.int32(0)
    m_t = np.int32(0)
    for t in range(NW):
        pltpu.sync_copy(counts.at[t], cntb)
        mt = cntb[pl.ds(0, 16)][0]
        b_t = b_t + jnp.where(np.int32(t) < wid, mt, 0)
        m_t = m_t + jnp.where(np.int32(t) == wid, mt, 0)

    pltpu.sync_copy(lists_rel.at[wid], lrel.at[pl.ds(0, LCAP)])
    pltpu.sync_copy(lists_src.at[wid], lsrc.at[pl.ds(0, LCAP)])

    for k in range(31):
        cuts_sm[k] = np.int32(_CUTS[k])

    zv = jnp.zeros((16,), jnp.float32)

    def zero_prow(i, _):
        for ch in range(8):
            pacc[i, pl.ds(ch * 16, 16)] = zv
        return 0

    lax.fori_loop(0, NLOC + 16, zero_prow, 0)

    def zero_sm(i, _):
        hist_sm[i] = np.int32(0)
        cnt_sm[i] = np.int32(0)
        return 0

    lax.fori_loop(0, NLOC + 16, zero_sm, 0)

    def hist_body(e, _):
        slot = lrel[pl.ds(e, 16)][0]
        hist_sm[slot] = hist_sm[slot] + 1
        return 0

    lax.fori_loop(0, m_t, hist_body, 0)

    def brk_body(i, carry):
        running, nbd = carry
        h = hist_sm[i]
        ss = b_t + running
        se = ss + h
        br = BIG
        for k in range(31):
            c = cuts_sm[k]
            hit = (c > ss) & (c < se)
            br = jnp.where(hit & (br == BIG), c - ss, br)
        has = (br != BIG).astype(jnp.int32)
        brk_sm[i] = br
        bds_sm[i] = np.int32(NLOC) + nbd
        return (running + h, nbd + has)

    lax.fori_loop(0, NLOC + 16, brk_body, (np.int32(0), np.int32(0)))

    nch = (m_t + (CH - 1)) // CH

    def chunk_body(c, _):
        pltpu.async_copy(x_hbm.at[lsrc.at[pl.ds(c * CH, CH)]], gbuf, sem).wait()

        def grp_body(g, _):
            slots = lrel[pl.ds(c * CH + g * 16, 16)]
            for l in range(16):
                slot = slots[l]
                r = cnt_sm[slot]
                cnt_sm[slot] = r + 1
                tgt = jnp.where(r >= brk_sm[slot], bds_sm[slot], slot)
                for ch in range(8):
                    plsc.addupdate(pacc.at[tgt, pl.ds(ch * 16, 16)],
                                   gbuf[g * 16 + l, pl.ds(ch * 16, 16)])
            return 0

        lax.fori_loop(0, CH // 16, grp_body, 0)
        return 0

    lax.fori_loop(0, nch, chunk_body, 0)

    def post_body(i, _):
        br = brk_sm[i]

        @pl.when(br != BIG)
        def _():
            bslot = bds_sm[i]
            for ch in range(8):
                plsc.addupdate(pacc.at[i, pl.ds(ch * 16, 16)],
                               pacc[bslot, pl.ds(ch * 16, 16)])

        return 0

    lax.fori_loop(0, NLOC, post_body, 0)

    pltpu.sync_copy(pacc.at[pl.ds(0, NLOC)], agg_hbm.at[pl.ds(base, NLOC)])


def _sc_compact(dst, src):
    mesh = plsc.VectorSubcoreMesh(core_axis_name="c", subcore_axis_name="s")
    return pl.kernel(
        _scan_body,
        out_type=(
            jax.ShapeDtypeStruct((NW, LCAP), jnp.int32),
            jax.ShapeDtypeStruct((NW, LCAP), jnp.int32),
            jax.ShapeDtypeStruct((NW, 128), jnp.int32),
        ),
        mesh=mesh,
        compiler_params=pltpu.CompilerParams(needs_layout_passes=False),
        scratch_types=[
            pltpu.VMEM((WIN,), jnp.int32),
            pltpu.VMEM((WIN,), jnp.int32),
            pltpu.VMEM((LCAP + 16,), jnp.int32),
            pltpu.VMEM((LCAP + 16,), jnp.int32),
        ],
    )(dst, src)


def _sc_accumulate(lists_rel, lists_src, counts, x):
    mesh = plsc.VectorSubcoreMesh(core_axis_name="c", subcore_axis_name="s")
    return pl.kernel(
        _acc_body,
        out_type=jax.ShapeDtypeStruct((NPAD, D), jnp.float32),
        mesh=mesh,
        compiler_params=pltpu.CompilerParams(needs_layout_passes=False),
        scratch_types=[
            pltpu.VMEM((LCAP + 16,), jnp.int32),
            pltpu.VMEM((LCAP + 16,), jnp.int32),
            pltpu.VMEM((CH, D), jnp.float32),
            pltpu.VMEM((NLOC + 16, D), jnp.float32),
            pltpu.VMEM((128,), jnp.int32),
            pltpu.SMEM((NLOC + 16,), jnp.int32),
            pltpu.SMEM((NLOC + 16,), jnp.int32),
            pltpu.SMEM((NLOC + 16,), jnp.int32),
            pltpu.SMEM((NLOC + 16,), jnp.int32),
            pltpu.SMEM((32,), jnp.int32),
            pltpu.SemaphoreType.DMA,
        ],
    )(lists_rel, lists_src, counts, x)


def _score_body(a_ref, x_ref, w_ref, b_ref, ws_ref, nrm_ref, o_ref):
    dims = (((1,), (0,)), ((), ()))
    d1 = lax.dot_general(a_ref[...], w_ref[...], dims,
                         preferred_element_type=jnp.float32)[:, 0:1]
    d2 = lax.dot_general(x_ref[...], w_ref[...], dims,
                         preferred_element_type=jnp.float32)[:, 1:2]
    attn = (d1 + b_ref[0, 0]) + d2
    o_ref[...] = jnp.tanh((attn * ws_ref[0, 0]) / nrm_ref[0, 0])


def _tc_score(agg_pad, x_pad, Wp, b_rel, w_sel, nrm):
    blk = 512
    grid = NPAD // blk
    return pl.pallas_call(
        _score_body,
        grid=(grid,),
        in_specs=[
            pl.BlockSpec((blk, D), lambda i: (i, 0)),
            pl.BlockSpec((blk, D), lambda i: (i, 0)),
            pl.BlockSpec((D, D), lambda i: (0, 0)),
            pl.BlockSpec((1, 1), lambda i: (0, 0)),
            pl.BlockSpec((1, 1), lambda i: (0, 0)),
            pl.BlockSpec((1, 1), lambda i: (0, 0)),
        ],
        out_specs=pl.BlockSpec((blk, 1), lambda i: (i, 0)),
        out_shape=jax.ShapeDtypeStruct((NPAD, 1), jnp.float32),
    )(agg_pad, x_pad, Wp, b_rel, w_sel, nrm)


def kernel(x, edge_index, W_rel, b_rel, W_root, w_sel):
    src = edge_index[0]
    dst = edge_index[1]

    lists_rel, lists_src, counts = _sc_compact(dst, src)
    agg_pad = _sc_accumulate(lists_rel, lists_src, counts, x)

    x_pad = jnp.zeros((NPAD, D), jnp.float32).at[:N].set(x)
    Wp = jnp.zeros((D, D), jnp.float32).at[:, 0].set(W_rel[0]).at[:, 1].set(W_root[0])
    nrm = jnp.linalg.norm(w_sel).reshape(1, 1)

    score = _tc_score(agg_pad, x_pad, Wp, b_rel.reshape(1, 1), w_sel, nrm)[:N, 0]
    vals, node_index = jax.lax.top_k(score, K)
    return node_index, vals


# speedup vs baseline: 1.1995x; 1.1995x over previous
import functools
import math

import jax
import jax.numpy as jnp
import numpy as np
from jax import lax
from jax.experimental import pallas as pl
from jax.experimental.pallas import tpu as pltpu
from jax.experimental.pallas import tpu_sc as plsc

N = 10000
E = 320000
D = 128
K = 5000

NW = 32
NLOC = 320
NPAD = NW * NLOC
LCAP = 12800
WIN = 2000
NVEC = WIN // 16
NWIN = E // WIN
CH = 128
BIG = np.int32(2**30)

_HALF = [10080 * k for k in range(1, 12)] + [120720, 130560, 140400, 150240]
_CUTS = _HALF + [160000] + [160000 + c for c in _HALF]


def _scan_body(dst_hbm, src_hbm, lists_rel, lists_src, counts, dstb, srcb,
               lrel, lsrc):
    wid = lax.axis_index("s") * 2 + lax.axis_index("c")
    base = wid * NLOC
    nloc = jnp.minimum(np.int32(NLOC), np.int32(N) - base)
    lane = lax.iota(jnp.int32, 16)

    trash_v = jnp.zeros((16,), jnp.int32) + np.int32(NLOC + 15)

    def zero_body(j, _):
        lrel[pl.ds(j * 16, 16)] = trash_v
        lsrc[pl.ds(j * 16, 16)] = jnp.zeros((16,), jnp.int32)
        return 0

    lax.fori_loop(0, (LCAP + 16) // 16, zero_body, 0)

    def win_body(w, cur):
        pltpu.sync_copy(dst_hbm.at[pl.ds(w * WIN, WIN)], dstb)
        pltpu.sync_copy(src_hbm.at[pl.ds(w * WIN, WIN)], srcb)

        def vec_body(j, cur):
            dv = dstb[pl.ds(j * 16, 16)]
            sv = srcb[pl.ds(j * 16, 16)]
            rel = dv - base
            m = (rel >= 0) & (rel < nloc)
            mi = jnp.where(m, 1, 0).astype(jnp.int32)
            cs = plsc.cumsum(mi)
            pos = jnp.where(m, cur + cs - 1, LCAP + lane)
            plsc.store_scatter(lrel, [pos], rel)
            plsc.store_scatter(lsrc, [pos], sv)
            return cur + cs[15]

        return lax.fori_loop(0, NVEC, vec_body, cur)

    m_t = lax.fori_loop(0, NWIN, win_body, np.int32(0))

    pltpu.sync_copy(lrel.at[pl.ds(0, LCAP)], lists_rel.at[wid])
    pltpu.sync_copy(lsrc.at[pl.ds(0, LCAP)], lists_src.at[wid])
    for j in range(8):
        dstb[pl.ds(j * 16, 16)] = jnp.zeros((16,), jnp.int32) + m_t
    pltpu.sync_copy(dstb.at[pl.ds(0, 128)], counts.at[wid])


def _acc_body(lists_rel, lists_src, counts, x_hbm, agg_hbm, lrel, lsrc, gbuf,
              pacc, cntb, hist_sm, cnt_sm, brk_sm, bds_sm, cuts_sm, sem):
    wid = lax.axis_index("s") * 2 + lax.axis_index("c")
    base = wid * NLOC

    b_t = np.int32(0)
    m_t = np.int32(0)
    for t in range(NW):
        pltpu.sync_copy(counts.at[t], cntb)
        mt = cntb[pl.ds(0, 16)][0]
        b_t = b_t + jnp.where(np.int32(t) < wid, mt, 0)
        m_t = m_t + jnp.where(np.int32(t) == wid, mt, 0)

    pltpu.sync_copy(lists_rel.at[wid], lrel.at[pl.ds(0, LCAP)])
    pltpu.sync_copy(lists_src.at[wid], lsrc.at[pl.ds(0, LCAP)])

    for k in range(31):
        cuts_sm[k] = np.int32(_CUTS[k])

    zv = jnp.zeros((16,), jnp.float32)

    def zero_prow(i, _):
        for ch in range(8):
            pacc[i, pl.ds(ch * 16, 16)] = zv
        return 0

    lax.fori_loop(0, NLOC + 16, zero_prow, 0)

    def zero_sm(i, _):
        hist_sm[i] = np.int32(0)
        cnt_sm[i] = np.int32(0)
        return 0

    lax.fori_loop(0, NLOC + 16, zero_sm, 0)

    def hist_body(e, _):
        slot = lrel[pl.ds(e, 16)][0]
        hist_sm[slot] = hist_sm[slot] + 1
        return 0

    lax.fori_loop(0, m_t, hist_body, 0)

    def brk_body(i, carry):
        running, nbd = carry
        h = hist_sm[i]
        ss = b_t + running
        se = ss + h
        br = BIG
        for k in range(31):
            c = cuts_sm[k]
            hit = (c > ss) & (c < se)
            br = jnp.where(hit & (br == BIG), c - ss, br)
        has = (br != BIG).astype(jnp.int32)
        brk_sm[i] = br
        bds_sm[i] = np.int32(NLOC) + nbd
        return (running + h, nbd + has)

    lax.fori_loop(0, NLOC + 16, brk_body, (np.int32(0), np.int32(0)))

    nch = (m_t + (CH - 1)) // CH

    def chunk_body(c, _):
        pltpu.async_copy(x_hbm.at[lsrc.at[pl.ds(c * CH, CH)]], gbuf, sem).wait()

        def grp_body(g, _):
            slots = lrel[pl.ds(c * CH + g * 16, 16)]
            for l in range(16):
                slot = slots[l]
                r = cnt_sm[slot]
                cnt_sm[slot] = r + 1
                tgt = jnp.where(r >= brk_sm[slot], bds_sm[slot], slot)
                for ch in range(8):
                    plsc.addupdate(pacc.at[tgt, pl.ds(ch * 16, 16)],
                                   gbuf[g * 16 + l, pl.ds(ch * 16, 16)])
            return 0

        lax.fori_loop(0, CH // 16, grp_body, 0)
        return 0

    lax.fori_loop(0, nch, chunk_body, 0)

    def post_body(i, _):
        br = brk_sm[i]

        @pl.when(br != BIG)
        def _():
            bslot = bds_sm[i]
            for ch in range(8):
                plsc.addupdate(pacc.at[i, pl.ds(ch * 16, 16)],
                               pacc[bslot, pl.ds(ch * 16, 16)])

        return 0

    lax.fori_loop(0, NLOC, post_body, 0)

    pltpu.sync_copy(pacc.at[pl.ds(0, NLOC)], agg_hbm.at[pl.ds(base, NLOC)])


def _sc_compact(dst, src):
    mesh = plsc.VectorSubcoreMesh(core_axis_name="c", subcore_axis_name="s")
    return pl.kernel(
        _scan_body,
        out_type=(
            jax.ShapeDtypeStruct((NW, LCAP), jnp.int32),
            jax.ShapeDtypeStruct((NW, LCAP), jnp.int32),
            jax.ShapeDtypeStruct((NW, 128), jnp.int32),
        ),
        mesh=mesh,
        compiler_params=pltpu.CompilerParams(needs_layout_passes=False),
        scratch_types=[
            pltpu.VMEM((WIN,), jnp.int32),
            pltpu.VMEM((WIN,), jnp.int32),
            pltpu.VMEM((LCAP + 16,), jnp.int32),
            pltpu.VMEM((LCAP + 16,), jnp.int32),
        ],
    )(dst, src)


def _sc_accumulate(lists_rel, lists_src, counts, x):
    mesh = plsc.VectorSubcoreMesh(core_axis_name="c", subcore_axis_name="s")
    return pl.kernel(
        _acc_body,
        out_type=jax.ShapeDtypeStruct((NPAD, D), jnp.float32),
        mesh=mesh,
        compiler_params=pltpu.CompilerParams(needs_layout_passes=False),
        scratch_types=[
            pltpu.VMEM((LCAP + 16,), jnp.int32),
            pltpu.VMEM((LCAP + 16,), jnp.int32),
            pltpu.VMEM((CH, D), jnp.float32),
            pltpu.VMEM((NLOC + 16, D), jnp.float32),
            pltpu.VMEM((128,), jnp.int32),
            pltpu.SMEM((NLOC + 16,), jnp.int32),
            pltpu.SMEM((NLOC + 16,), jnp.int32),
            pltpu.SMEM((NLOC + 16,), jnp.int32),
            pltpu.SMEM((NLOC + 16,), jnp.int32),
            pltpu.SMEM((32,), jnp.int32),
            pltpu.SemaphoreType.DMA,
        ],
    )(lists_rel, lists_src, counts, x)


def _score_body(a_ref, x_ref, w_ref, b_ref, ws_ref, nrm_ref, o_ref):
    dims = (((1,), (0,)), ((), ()))
    d1 = lax.dot_general(a_ref[...], w_ref[...], dims,
                         preferred_element_type=jnp.float32)[:, 0:1]
    d2 = lax.dot_general(x_ref[...], w_ref[...], dims,
                         preferred_element_type=jnp.float32)[:, 1:2]
    attn = (d1 + b_ref[0, 0]) + d2
    o_ref[...] = jnp.tanh((attn * ws_ref[0, 0]) / nrm_ref[0, 0])


def _tc_score(agg_pad, x_pad, Wp, b_rel, w_sel, nrm):
    blk = 512
    grid = NPAD // blk
    return pl.pallas_call(
        _score_body,
        grid=(grid,),
        in_specs=[
            pl.BlockSpec((blk, D), lambda i: (i, 0)),
            pl.BlockSpec((blk, D), lambda i: (i, 0)),
            pl.BlockSpec((D, D), lambda i: (0, 0)),
            pl.BlockSpec((1, 1), lambda i: (0, 0)),
            pl.BlockSpec((1, 1), lambda i: (0, 0)),
            pl.BlockSpec((1, 1), lambda i: (0, 0)),
        ],
        out_specs=pl.BlockSpec((blk, 1), lambda i: (i, 0)),
        out_shape=jax.ShapeDtypeStruct((NPAD, 1), jnp.float32),
    )(agg_pad, x_pad, Wp, b_rel, w_sel, nrm)


def kernel(x, edge_index, W_rel, b_rel, W_root, w_sel):
    src = edge_index[0]
    dst = edge_index[1]

    lists_rel, lists_src, counts = _sc_compact(dst, src)
    agg_pad = _sc_accumulate(lists_rel, lists_src, counts, x)

    x_pad = jnp.zeros((NPAD, D), jnp.float32).at[:N].set(x)
    Wp = jnp.zeros((D, D), jnp.float32).at[:, 0].set(W_rel[0]).at[:, 1].set(W_root[0])
    nrm = jnp.linalg.norm(w_sel).reshape(1, 1)

    score = _tc_score(agg_pad, x_pad, Wp, b_rel.reshape(1, 1), w_sel, nrm)[:N, 0]
    vals, node_index = jax.lax.top_k(score, K)
    return node_index, vals


# speedup vs baseline: 1.6508x; 1.3763x over previous
import functools
import math

import jax
import jax.numpy as jnp
import numpy as np
from jax import lax
from jax.experimental import pallas as pl
from jax.experimental.pallas import tpu as pltpu
from jax.experimental.pallas import tpu_sc as plsc

N = 10000
E = 320000
D = 128
K = 5000

NW = 32
NLOC = 320
NPAD = NW * NLOC
LCAP = 12800
WIN = 2000
NVEC = WIN // 16
NWIN = E // WIN
CH = 128
BIG = np.int32(2**30)

_HALF = [10080 * k for k in range(1, 12)] + [120720, 130560, 140400, 150240]
_CUTS = _HALF + [160000] + [160000 + c for c in _HALF]


def _scan_body(dst_hbm, src_hbm, lists_rel, lists_src, counts, dstb, srcb,
               lrel, lsrc, sema):
    wid = lax.axis_index("s") * 2 + lax.axis_index("c")
    base = wid * NLOC
    nloc = jnp.minimum(np.int32(NLOC), np.int32(N) - base)
    lane = lax.iota(jnp.int32, 16)

    trash_v = jnp.zeros((16,), jnp.int32) + np.int32(NLOC + 15)

    def zero_body(j, _):
        lrel[pl.ds(j * 16, 16)] = trash_v
        lsrc[pl.ds(j * 16, 16)] = jnp.zeros((16,), jnp.int32)
        return 0

    lax.fori_loop(0, (LCAP + 16) // 16, zero_body, 0)

    cp = pltpu.async_copy(dst_hbm.at[pl.ds(0, WIN)], dstb.at[pl.ds(0, WIN)], sema)
    cp2 = pltpu.async_copy(src_hbm.at[pl.ds(0, WIN)], srcb.at[pl.ds(0, WIN)], sema)

    def win_body(w, cur):
        par = w % 2
        pltpu.make_async_copy(dst_hbm.at[pl.ds(w * WIN, WIN)],
                              dstb.at[pl.ds(par * WIN, WIN)], sema).wait()
        pltpu.make_async_copy(src_hbm.at[pl.ds(w * WIN, WIN)],
                              srcb.at[pl.ds(par * WIN, WIN)], sema).wait()

        @pl.when(w + 1 < NWIN)
        def _():
            pltpu.async_copy(dst_hbm.at[pl.ds((w + 1) * WIN, WIN)],
                             dstb.at[pl.ds((1 - par) * WIN, WIN)], sema)
            pltpu.async_copy(src_hbm.at[pl.ds((w + 1) * WIN, WIN)],
                             srcb.at[pl.ds((1 - par) * WIN, WIN)], sema)

        def vec_body(j, cur):
            dv = dstb[pl.ds(par * WIN + j * 16, 16)]
            sv = srcb[pl.ds(par * WIN + j * 16, 16)]
            rel = dv - base
            m = (rel >= 0) & (rel < nloc)
            mi = jnp.where(m, 1, 0).astype(jnp.int32)
            cs = plsc.cumsum(mi)
            pos = jnp.where(m, cur + cs - 1, LCAP + lane)
            plsc.store_scatter(lrel, [pos], rel)
            plsc.store_scatter(lsrc, [pos], sv)
            return cur + cs[15]

        return lax.fori_loop(0, NVEC, vec_body, cur)

    m_t = lax.fori_loop(0, NWIN, win_body, np.int32(0))

    pltpu.sync_copy(lrel.at[pl.ds(0, LCAP)], lists_rel.at[wid])
    pltpu.sync_copy(lsrc.at[pl.ds(0, LCAP)], lists_src.at[wid])
    for j in range(8):
        dstb[pl.ds(j * 16, 16)] = jnp.zeros((16,), jnp.int32) + m_t
    pltpu.sync_copy(dstb.at[pl.ds(0, 128)], counts.at[wid])


def _acc_body(lists_rel, lists_src, counts, x_hbm, agg_hbm, lrel, lsrc, gbuf,
              pacc, cntb, hist_sm, cnt_sm, brk_sm, bds_sm, cuts_sm, sem):
    wid = lax.axis_index("s") * 2 + lax.axis_index("c")
    base = wid * NLOC

    b_t = np.int32(0)
    m_t = np.int32(0)
    for t in range(NW):
        pltpu.sync_copy(counts.at[t], cntb)
        mt = cntb[pl.ds(0, 16)][0]
        b_t = b_t + jnp.where(np.int32(t) < wid, mt, 0)
        m_t = m_t + jnp.where(np.int32(t) == wid, mt, 0)

    pltpu.sync_copy(lists_rel.at[wid], lrel.at[pl.ds(0, LCAP)])
    pltpu.sync_copy(lists_src.at[wid], lsrc.at[pl.ds(0, LCAP)])

    for k in range(31):
        cuts_sm[k] = np.int32(_CUTS[k])

    zv = jnp.zeros((16,), jnp.float32)

    def zero_prow(i, _):
        for ch in range(8):
            pacc[i, pl.ds(ch * 16, 16)] = zv
        return 0

    lax.fori_loop(0, NLOC + 16, zero_prow, 0)

    def zero_sm(i, _):
        hist_sm[i] = np.int32(0)
        cnt_sm[i] = np.int32(0)
        return 0

    lax.fori_loop(0, NLOC + 16, zero_sm, 0)

    def hist_body(g, _):
        slots = lrel[pl.ds(g * 16, 16)]
        for l in range(16):
            slot = slots[l]
            hist_sm[slot] = hist_sm[slot] + 1
        return 0

    lax.fori_loop(0, (m_t + 15) // 16, hist_body, 0)

    def clear_trash(i, _):
        hist_sm[NLOC + i] = np.int32(0)
        return 0

    lax.fori_loop(0, 16, clear_trash, 0)

    def brk_body(i, carry):
        running, nbd = carry
        h = hist_sm[i]
        ss = b_t + running
        se = ss + h
        br = BIG
        for k in range(31):
            c = cuts_sm[k]
            hit = (c > ss) & (c < se)
            br = jnp.where(hit & (br == BIG), c - ss, br)
        has = (br != BIG).astype(jnp.int32)
        brk_sm[i] = br
        bds_sm[i] = np.int32(NLOC) + nbd
        return (running + h, nbd + has)

    lax.fori_loop(0, NLOC + 16, brk_body, (np.int32(0), np.int32(0)))

    nch = (m_t + (CH - 1)) // CH

    @pl.when(nch > 0)
    def _():
        pltpu.async_copy(x_hbm.at[lsrc.at[pl.ds(0, CH)]], gbuf.at[pl.ds(0, CH)],
                         sem)

    def chunk_body(c, _):
        par = c % 2
        pltpu.make_async_copy(x_hbm.at[lsrc.at[pl.ds(c * CH, CH)]],
                              gbuf.at[pl.ds(par * CH, CH)], sem).wait()

        @pl.when(c + 1 < nch)
        def _():
            pltpu.async_copy(x_hbm.at[lsrc.at[pl.ds((c + 1) * CH, CH)]],
                             gbuf.at[pl.ds((1 - par) * CH, CH)], sem)

        def grp_body(g, _):
            slots = lrel[pl.ds(c * CH + g * 16, 16)]
            for l in range(16):
                slot = slots[l]
                r = cnt_sm[slot]
                cnt_sm[slot] = r + 1
                tgt = jnp.where(r >= brk_sm[slot], bds_sm[slot], slot)
                for ch in range(8):
                    plsc.addupdate(pacc.at[tgt, pl.ds(ch * 16, 16)],
                                   gbuf[par * CH + g * 16 + l, pl.ds(ch * 16, 16)])
            return 0

        lax.fori_loop(0, CH // 16, grp_body, 0)
        return 0

    lax.fori_loop(0, nch, chunk_body, 0)

    def post_body(i, _):
        br = brk_sm[i]

        @pl.when(br != BIG)
        def _():
            bslot = bds_sm[i]
            for ch in range(8):
                plsc.addupdate(pacc.at[i, pl.ds(ch * 16, 16)],
                               pacc[bslot, pl.ds(ch * 16, 16)])

        return 0

    lax.fori_loop(0, NLOC, post_body, 0)

    pltpu.sync_copy(pacc.at[pl.ds(0, NLOC)], agg_hbm.at[pl.ds(base, NLOC)])


def _sc_compact(dst, src):
    mesh = plsc.VectorSubcoreMesh(core_axis_name="c", subcore_axis_name="s")
    return pl.kernel(
        _scan_body,
        out_type=(
            jax.ShapeDtypeStruct((NW, LCAP), jnp.int32),
            jax.ShapeDtypeStruct((NW, LCAP), jnp.int32),
            jax.ShapeDtypeStruct((NW, 128), jnp.int32),
        ),
        mesh=mesh,
        compiler_params=pltpu.CompilerParams(needs_layout_passes=False),
        scratch_types=[
            pltpu.VMEM((2 * WIN,), jnp.int32),
            pltpu.VMEM((2 * WIN,), jnp.int32),
            pltpu.VMEM((LCAP + 16,), jnp.int32),
            pltpu.VMEM((LCAP + 16,), jnp.int32),
            pltpu.SemaphoreType.DMA,
        ],
    )(dst, src)


def _sc_accumulate(lists_rel, lists_src, counts, x):
    mesh = plsc.VectorSubcoreMesh(core_axis_name="c", subcore_axis_name="s")
    return pl.kernel(
        _acc_body,
        out_type=jax.ShapeDtypeStruct((NPAD, D), jnp.float32),
        mesh=mesh,
        compiler_params=pltpu.CompilerParams(needs_layout_passes=False),
        scratch_types=[
            pltpu.VMEM((LCAP + 16,), jnp.int32),
            pltpu.VMEM((LCAP + 16,), jnp.int32),
            pltpu.VMEM((2 * CH, D), jnp.float32),
            pltpu.VMEM((NLOC + 16, D), jnp.float32),
            pltpu.VMEM((128,), jnp.int32),
            pltpu.SMEM((NLOC + 16,), jnp.int32),
            pltpu.SMEM((NLOC + 16,), jnp.int32),
            pltpu.SMEM((NLOC + 16,), jnp.int32),
            pltpu.SMEM((NLOC + 16,), jnp.int32),
            pltpu.SMEM((32,), jnp.int32),
            pltpu.SemaphoreType.DMA,
        ],
    )(lists_rel, lists_src, counts, x)


def _score_body(a_ref, x_ref, w_ref, b_ref, ws_ref, nrm_ref, o_ref):
    dims = (((1,), (0,)), ((), ()))
    d1 = lax.dot_general(a_ref[...], w_ref[...], dims,
                         preferred_element_type=jnp.float32)[:, 0:1]
    d2 = lax.dot_general(x_ref[...], w_ref[...], dims,
                         preferred_element_type=jnp.float32)[:, 1:2]
    attn = (d1 + b_ref[0, 0]) + d2
    o_ref[...] = jnp.tanh((attn * ws_ref[0, 0]) / nrm_ref[0, 0])


def _tc_score(agg_pad, x_pad, Wp, b_rel, w_sel, nrm):
    blk = 512
    grid = NPAD // blk
    return pl.pallas_call(
        _score_body,
        grid=(grid,),
        in_specs=[
            pl.BlockSpec((blk, D), lambda i: (i, 0)),
            pl.BlockSpec((blk, D), lambda i: (i, 0)),
            pl.BlockSpec((D, D), lambda i: (0, 0)),
            pl.BlockSpec((1, 1), lambda i: (0, 0)),
            pl.BlockSpec((1, 1), lambda i: (0, 0)),
            pl.BlockSpec((1, 1), lambda i: (0, 0)),
        ],
        out_specs=pl.BlockSpec((blk, 1), lambda i: (i, 0)),
        out_shape=jax.ShapeDtypeStruct((NPAD, 1), jnp.float32),
    )(agg_pad, x_pad, Wp, b_rel, w_sel, nrm)


def kernel(x, edge_index, W_rel, b_rel, W_root, w_sel):
    src = edge_index[0]
    dst = edge_index[1]

    lists_rel, lists_src, counts = _sc_compact(dst, src)
    agg_pad = _sc_accumulate(lists_rel, lists_src, counts, x)

    x_pad = jnp.zeros((NPAD, D), jnp.float32).at[:N].set(x)
    Wp = jnp.zeros((D, D), jnp.float32).at[:, 0].set(W_rel[0]).at[:, 1].set(W_root[0])
    nrm = jnp.linalg.norm(w_sel).reshape(1, 1)

    score = _tc_score(agg_pad, x_pad, Wp, b_rel.reshape(1, 1), w_sel, nrm)[:N, 0]
    vals, node_index = jax.lax.top_k(score, K)
    return node_index, vals
